# baseline (speedup 1.0000x reference)
import jax
import jax.numpy as jnp
import numpy as np
from jax.experimental import pallas as pl

_R = 32
_N = 32768
_NC = 2048
_CK = _N // _NC
_GT = _NC // 128
_K = 128

_LOW = np.int32(0x7FFFFFFF)


def _to_key(score):
    bits = jax.lax.bitcast_convert_type(score, jnp.int32)
    return jnp.where(bits < 0, bits ^ _LOW, bits)


def _from_key(key):
    bits = jnp.where(key < 0, key ^ _LOW, key)
    return jax.lax.bitcast_convert_type(bits, jnp.float32)


def _lex_gt(k_a, c_a, k_b, c_b):
    return (k_a > k_b) | ((k_a == k_b) & (c_a < c_b))


def _cmpex(skey, scol, d, desc, lane):
    upper = (lane & d) == 0
    pk = jnp.where(upper, jnp.roll(skey, -d, axis=-1),
                   jnp.roll(skey, d, axis=-1))
    pc = jnp.where(upper, jnp.roll(scol, -d, axis=-1),
                   jnp.roll(scol, d, axis=-1))
    p_better = _lex_gt(pk, pc, skey, scol)
    want_max = upper == desc
    take = p_better ^ ~want_max
    return jnp.where(take, pk, skey), jnp.where(take, pc, scol)


def _sort_merge(skey, scol):
    lane3 = jax.lax.broadcasted_iota(jnp.int32, (1, 1, _K), 2)
    half = jax.lax.broadcasted_iota(jnp.int32, (1, _CK, 1), 1) < (_CK // 2)
    for s in range(1, 8):
        desc = (((lane3 >> s) & 1) == 0) == half
        for j in range(s - 1, -1, -1):
            skey, scol = _cmpex(skey, scol, 1 << j, desc, lane3)
    h = _CK // 2
    while h >= 1:
        ak, bk = skey[:, :h, :], skey[:, h:, :]
        ac, bc = scol[:, :h, :], scol[:, h:, :]
        b_wins = _lex_gt(bk, bc, ak, ac)
        skey = jnp.where(b_wins, bk, ak)
        scol = jnp.where(b_wins, bc, ac)
        if h > 1:
            desc = jax.lax.broadcasted_iota(jnp.int32, (1, h, 1), 1) < (h // 2)
        else:
            desc = jnp.ones((1, 1, 1), dtype=bool)
        for j in range(6, -1, -1):
            skey, scol = _cmpex(skey, scol, 1 << j, desc, lane3)
        h //= 2
    return skey[:, 0, :], scol[:, 0, :]


def _topk_body(mean_ref, var_ref, yb_ref, vals_ref, idx_ref, masked_ref):
    mean = mean_ref[...]
    var = var_ref[...]
    yb = yb_ref[...]

    sigma = jnp.sqrt(var + 1e-6)
    u = (mean - yb) / sigma
    Phi = 0.5 * (1.0 + jax.lax.erf(u / jnp.sqrt(2.0).astype(jnp.float32)))
    phi = jnp.exp(-0.5 * u * u) / jnp.sqrt(2.0 * jnp.pi).astype(jnp.float32)
    score = sigma * (u * Phi + phi)

    key = _to_key(score)
    del score

    lane2 = jax.lax.broadcasted_iota(jnp.int32, (_R, 128), 1)
    m_tiles, am_tiles = [], []
    for g in range(_GT):
        mg = key[:, g * 128:(g + 1) * 128]
        amg = lane2 + g * 128
        for k in range(1, _CK):
            xs = key[:, k * _NC + g * 128:k * _NC + (g + 1) * 128]
            upd = xs > mg
            amg = jnp.where(upd, lane2 + (g * 128 + k * _NC), amg)
            mg = jnp.where(upd, xs, mg)
        m_tiles.append(mg)
        am_tiles.append(amg)
    m3 = jnp.stack(m_tiles, axis=1)
    am3 = jnp.stack(am_tiles, axis=1)

    _, amtop = _sort_merge(m3, am3)
    ctop = amtop & (_NC - 1)
    hi = ctop >> 7
    lo = ctop & 127

    cand_ks = []
    for k in range(_CK):
        acc = jnp.zeros((_R, _K), jnp.int32)
        for g in range(_GT):
            src = key[:, k * _NC + g * 128:k * _NC + (g + 1) * 128]
            gath = jnp.take_along_axis(src, lo, axis=1)
            acc = jnp.where(hi == g, gath, acc)
        cand_ks.append(acc)
    key2 = jnp.stack(cand_ks, axis=1)
    col2 = jnp.stack([ctop + k * _NC for k in range(_CK)], axis=1)

    skey2, scol2 = _sort_merge(key2, col2)

    vals_ref[...] = _from_key(skey2)
    idx_ref[...] = scol2

    colfull = jax.lax.broadcasted_iota(jnp.int32, (_R, _N), 1)
    Vfull = skey2[:, _K - 1:_K]
    cfull = scol2[:, _K - 1:_K]
    selfull = (key > Vfull) | ((key == Vfull) & (colfull <= cfull))
    masked_ref[...] = jnp.where(selfull, _from_key(key), -jnp.inf)


def kernel(mean, variance, y_best, q):
    R, N = mean.shape
    yb2 = y_best[:, None]
    grid = R // _R
    vals, idx, masked = pl.pallas_call(
        _topk_body,
        grid=(grid,),
        in_specs=[
            pl.BlockSpec((_R, N), lambda i: (i, 0)),
            pl.BlockSpec((_R, N), lambda i: (i, 0)),
            pl.BlockSpec((_R, 1), lambda i: (i, 0)),
        ],
        out_specs=[
            pl.BlockSpec((_R, _K), lambda i: (i, 0)),
            pl.BlockSpec((_R, _K), lambda i: (i, 0)),
            pl.BlockSpec((_R, N), lambda i: (i, 0)),
        ],
        out_shape=[
            jax.ShapeDtypeStruct((R, _K), jnp.float32),
            jax.ShapeDtypeStruct((R, _K), jnp.int32),
            jax.ShapeDtypeStruct((R, N), jnp.float32),
        ],
    )(mean, variance, yb2)
    return (vals, idx, masked)


# speedup vs baseline: 1.2510x; 1.2510x over previous
import jax
import jax.numpy as jnp
import numpy as np
from jax.experimental import pallas as pl

_R = 16
_N = 32768
_NC = 2048
_CK = _N // _NC
_GT = _NC // 128
_K = 128

_LOW = np.int32(0x7FFFFFFF)


def _to_key(score):
    bits = jax.lax.bitcast_convert_type(score, jnp.int32)
    return jnp.where(bits < 0, bits ^ _LOW, bits)


def _from_key(key):
    bits = jnp.where(key < 0, key ^ _LOW, key)
    return jax.lax.bitcast_convert_type(bits, jnp.float32)


def _lex_gt(k_a, c_a, k_b, c_b):
    return (k_a > k_b) | ((k_a == k_b) & (c_a < c_b))


def _cmpex(skey, scol, d, desc, lane):
    upper = (lane & d) == 0
    pk = jnp.where(upper, jnp.roll(skey, -d, axis=-1),
                   jnp.roll(skey, d, axis=-1))
    pc = jnp.where(upper, jnp.roll(scol, -d, axis=-1),
                   jnp.roll(scol, d, axis=-1))
    p_better = _lex_gt(pk, pc, skey, scol)
    want_max = upper == desc
    take = p_better ^ ~want_max
    return jnp.where(take, pk, skey), jnp.where(take, pc, scol)


def _sort_merge(skey, scol):
    lane3 = jax.lax.broadcasted_iota(jnp.int32, (1, 1, _K), 2)
    half = jax.lax.broadcasted_iota(jnp.int32, (1, _CK, 1), 1) < (_CK // 2)
    for s in range(1, 8):
        desc = (((lane3 >> s) & 1) == 0) == half
        for j in range(s - 1, -1, -1):
            skey, scol = _cmpex(skey, scol, 1 << j, desc, lane3)
    h = _CK // 2
    while h >= 1:
        ak, bk = skey[:, :h, :], skey[:, h:, :]
        ac, bc = scol[:, :h, :], scol[:, h:, :]
        b_wins = _lex_gt(bk, bc, ak, ac)
        skey = jnp.where(b_wins, bk, ak)
        scol = jnp.where(b_wins, bc, ac)
        if h > 1:
            desc = jax.lax.broadcasted_iota(jnp.int32, (1, h, 1), 1) < (h // 2)
        else:
            desc = jnp.ones((1, 1, 1), dtype=bool)
        for j in range(6, -1, -1):
            skey, scol = _cmpex(skey, scol, 1 << j, desc, lane3)
        h //= 2
    return skey[:, 0, :], scol[:, 0, :]


def _topk_body(mean_ref, var_ref, yb_ref, vals_ref, idx_ref, masked_ref):
    mean = mean_ref[...]
    var = var_ref[...]
    yb = yb_ref[...]

    sigma = jnp.sqrt(var + 1e-6)
    u = (mean - yb) / sigma
    Phi = 0.5 * (1.0 + jax.lax.erf(u / jnp.sqrt(2.0).astype(jnp.float32)))
    phi = jnp.exp(-0.5 * u * u) / jnp.sqrt(2.0 * jnp.pi).astype(jnp.float32)
    score = sigma * (u * Phi + phi)

    key = _to_key(score)
    del score

    lane2 = jax.lax.broadcasted_iota(jnp.int32, (_R, 128), 1)
    m_tiles, am_tiles = [], []
    for g in range(_GT):
        mg = key[:, g * 128:(g + 1) * 128]
        amg = lane2 + g * 128
        for k in range(1, _CK):
            xs = key[:, k * _NC + g * 128:k * _NC + (g + 1) * 128]
            upd = xs > mg
            amg = jnp.where(upd, lane2 + (g * 128 + k * _NC), amg)
            mg = jnp.where(upd, xs, mg)
        m_tiles.append(mg)
        am_tiles.append(amg)
    m3 = jnp.stack(m_tiles, axis=1)
    am3 = jnp.stack(am_tiles, axis=1)

    _, amtop = _sort_merge(m3, am3)
    ctop = amtop & (_NC - 1)
    hi = ctop >> 7
    lo = ctop & 127

    cand_ks = []
    for k in range(_CK):
        acc = jnp.zeros((_R, _K), jnp.int32)
        for g in range(_GT):
            src = key[:, k * _NC + g * 128:k * _NC + (g + 1) * 128]
            gath = jnp.take_along_axis(src, lo, axis=1)
            acc = jnp.where(hi == g, gath, acc)
        cand_ks.append(acc)
    key2 = jnp.stack(cand_ks, axis=1)
    col2 = jnp.stack([ctop + k * _NC for k in range(_CK)], axis=1)

    skey2, scol2 = _sort_merge(key2, col2)

    vals_ref[...] = _from_key(skey2)
    idx_ref[...] = scol2

    colfull = jax.lax.broadcasted_iota(jnp.int32, (_R, _N), 1)
    Vfull = skey2[:, _K - 1:_K]
    cfull = scol2[:, _K - 1:_K]
    selfull = (key > Vfull) | ((key == Vfull) & (colfull <= cfull))
    masked_ref[...] = jnp.where(selfull, _from_key(key), -jnp.inf)


def kernel(mean, variance, y_best, q):
    R, N = mean.shape
    yb2 = y_best[:, None]
    grid = R // _R
    vals, idx, masked = pl.pallas_call(
        _topk_body,
        grid=(grid,),
        in_specs=[
            pl.BlockSpec((_R, N), lambda i: (i, 0)),
            pl.BlockSpec((_R, N), lambda i: (i, 0)),
            pl.BlockSpec((_R, 1), lambda i: (i, 0)),
        ],
        out_specs=[
            pl.BlockSpec((_R, _K), lambda i: (i, 0)),
            pl.BlockSpec((_R, _K), lambda i: (i, 0)),
            pl.BlockSpec((_R, N), lambda i: (i, 0)),
        ],
        out_shape=[
            jax.ShapeDtypeStruct((R, _K), jnp.float32),
            jax.ShapeDtypeStruct((R, _K), jnp.int32),
            jax.ShapeDtypeStruct((R, N), jnp.float32),
        ],
    )(mean, variance, yb2)
    return (vals, idx, masked)


# speedup vs baseline: 1.2594x; 1.0067x over previous
import jax
import jax.numpy as jnp
import numpy as np
from jax.experimental import pallas as pl

_R = 16
_N = 32768
_NC = 2048
_CK = _N // _NC
_GT = _NC // 128
_K = 128

_LOW = np.int32(0x7FFFFFFF)


def _to_key(score):
    bits = jax.lax.bitcast_convert_type(score, jnp.int32)
    return jnp.where(bits < 0, bits ^ _LOW, bits)


def _from_key(key):
    bits = jnp.where(key < 0, key ^ _LOW, key)
    return jax.lax.bitcast_convert_type(bits, jnp.float32)


def _lex_gt(k_a, c_a, k_b, c_b):
    return (k_a > k_b) | ((k_a == k_b) & (c_a < c_b))


def _cmpex(skey, scol, d, desc, lane):
    upper = (lane & d) == 0
    pk = jnp.where(upper, jnp.roll(skey, -d, axis=-1),
                   jnp.roll(skey, d, axis=-1))
    pc = jnp.where(upper, jnp.roll(scol, -d, axis=-1),
                   jnp.roll(scol, d, axis=-1))
    p_better = _lex_gt(pk, pc, skey, scol)
    want_max = upper == desc
    take = p_better ^ ~want_max
    return jnp.where(take, pk, skey), jnp.where(take, pc, scol)


def _sort_merge(skey, scol):
    lane3 = jax.lax.broadcasted_iota(jnp.int32, (1, 1, _K), 2)
    half = jax.lax.broadcasted_iota(jnp.int32, (1, _CK, 1), 1) < (_CK // 2)
    for s in range(1, 8):
        desc = (((lane3 >> s) & 1) == 0) == half
        for j in range(s - 1, -1, -1):
            skey, scol = _cmpex(skey, scol, 1 << j, desc, lane3)
    h = _CK // 2
    while h >= 1:
        ak, bk = skey[:, :h, :], skey[:, h:, :]
        ac, bc = scol[:, :h, :], scol[:, h:, :]
        b_wins = _lex_gt(bk, bc, ak, ac)
        skey = jnp.where(b_wins, bk, ak)
        scol = jnp.where(b_wins, bc, ac)
        if h > 1:
            desc = jax.lax.broadcasted_iota(jnp.int32, (1, h, 1), 1) < (h // 2)
        else:
            desc = jnp.ones((1, 1, 1), dtype=bool)
        for j in range(6, -1, -1):
            skey, scol = _cmpex(skey, scol, 1 << j, desc, lane3)
        h //= 2
    return skey[:, 0, :], scol[:, 0, :]


def _topk_body(mean_ref, var_ref, yb_ref, vals_ref, idx_ref, masked_ref):
    mean = mean_ref[...]
    var = var_ref[...]
    yb = yb_ref[...]

    sigma = jnp.sqrt(var + 1e-6)
    u = (mean - yb) / sigma
    Phi = 0.5 * (1.0 + jax.lax.erf(u / jnp.sqrt(2.0).astype(jnp.float32)))
    phi = jnp.exp(-0.5 * u * u) / jnp.sqrt(2.0 * jnp.pi).astype(jnp.float32)
    score = sigma * (u * Phi + phi)

    key = _to_key(score)

    lane2 = jax.lax.broadcasted_iota(jnp.int32, (_R, 128), 1)
    m_tiles, am_tiles = [], []
    for g in range(_GT):
        mg = key[:, g * 128:(g + 1) * 128]
        amg = lane2 + g * 128
        for k in range(1, _CK):
            xs = key[:, k * _NC + g * 128:k * _NC + (g + 1) * 128]
            upd = xs > mg
            amg = jnp.where(upd, lane2 + (g * 128 + k * _NC), amg)
            mg = jnp.where(upd, xs, mg)
        m_tiles.append(mg)
        am_tiles.append(amg)
    m3 = jnp.stack(m_tiles, axis=1)
    am3 = jnp.stack(am_tiles, axis=1)

    _, amtop = _sort_merge(m3, am3)
    ctop = amtop & (_NC - 1)
    hi = ctop >> 7
    lo = ctop & 127

    cand_ks = []
    for k in range(_CK):
        acc = jnp.zeros((_R, _K), jnp.int32)
        for g in range(_GT):
            src = key[:, k * _NC + g * 128:k * _NC + (g + 1) * 128]
            gath = jnp.take_along_axis(src, lo, axis=1)
            acc = jnp.where(hi == g, gath, acc)
        cand_ks.append(acc)
    key2 = jnp.stack(cand_ks, axis=1)
    col2 = jnp.stack([ctop + k * _NC for k in range(_CK)], axis=1)

    skey2, scol2 = _sort_merge(key2, col2)

    vals_ref[...] = _from_key(skey2)
    idx_ref[...] = scol2

    colfull = jax.lax.broadcasted_iota(jnp.int32, (_R, _N), 1)
    Vfull = skey2[:, _K - 1:_K]
    cfull = scol2[:, _K - 1:_K]
    selfull = (key > Vfull) | ((key == Vfull) & (colfull <= cfull))
    masked_ref[...] = jnp.where(selfull, score, -jnp.inf)


def kernel(mean, variance, y_best, q):
    R, N = mean.shape
    yb2 = y_best[:, None]
    grid = R // _R
    vals, idx, masked = pl.pallas_call(
        _topk_body,
        grid=(grid,),
        in_specs=[
            pl.BlockSpec((_R, N), lambda i: (i, 0)),
            pl.BlockSpec((_R, N), lambda i: (i, 0)),
            pl.BlockSpec((_R, 1), lambda i: (i, 0)),
        ],
        out_specs=[
            pl.BlockSpec((_R, _K), lambda i: (i, 0)),
            pl.BlockSpec((_R, _K), lambda i: (i, 0)),
            pl.BlockSpec((_R, N), lambda i: (i, 0)),
        ],
        out_shape=[
            jax.ShapeDtypeStruct((R, _K), jnp.float32),
            jax.ShapeDtypeStruct((R, _K), jnp.int32),
            jax.ShapeDtypeStruct((R, N), jnp.float32),
        ],
    )(mean, variance, yb2)
    return (vals, idx, masked)
